# baseline (speedup 1.0000x reference)
import functools

import jax
import jax.numpy as jnp
from jax import lax
from jax.experimental import pallas as pl
from jax.experimental.pallas import tpu as pltpu
from jax.experimental.pallas import tpu_sc as plsc

_B, _S, _D = 4, 2048, 768
_NC, _NS = 2, 16
_NW = _NC * _NS
_PPW = _S // _NW
_CH = 32
_NCHUNK = _B * (_PPW // _CH)
_LPR = _D // 16

_mesh = plsc.VectorSubcoreMesh(core_axis_name="c", subcore_axis_name="s")


@functools.partial(
    pl.kernel,
    mesh=_mesh,
    out_type=jax.ShapeDtypeStruct((_B, _S, _D), jnp.float32),
    scratch_types=[
        pltpu.VMEM((_B, _PPW), jnp.int32),
        pltpu.VMEM((_PPW, _D), jnp.float32),
        pltpu.VMEM((3, _CH, _D), jnp.float32),
        pltpu.SemaphoreType.DMA((2,)),
        pltpu.SemaphoreType.DMA((3,)),
        pltpu.SemaphoreType.DMA((3,)),
    ],
)
def _emb_kernel(x_hbm, wte_hbm, wpe_hbm, out_hbm,
                idx_v, wpe_v, tok_v,
                sem_in, gsems, osems):
    gsem = tuple(gsems.at[i] for i in range(3))
    osem = tuple(osems.at[i] for i in range(3))
    wid = lax.axis_index("s") * _NC + lax.axis_index("c")
    pos0 = wid * _PPW

    idx_copies = [
        pltpu.async_copy(x_hbm.at[b, pl.ds(pos0, _PPW)], idx_v.at[b],
                         sem_in.at[0])
        for b in range(_B)
    ]
    wpe_copy = pltpu.async_copy(
        wpe_hbm.at[pl.ds(pos0, _PPW)], wpe_v, sem_in.at[1])
    for c in idx_copies:
        c.wait()

    def gather_start(c, bi):
        b, half = c // 2, c % 2
        return pltpu.async_copy(
            wte_hbm.at[idx_v.at[b, pl.ds(half * _CH, _CH)]],
            tok_v.at[bi], gsem[bi])

    def store_start(c, bi):
        b, half = c // 2, c % 2
        return pltpu.async_copy(
            tok_v.at[bi],
            out_hbm.at[b, pl.ds(pos0 + half * _CH, _CH)], osem[bi])

    def add_chunk(bi, half):
        @plsc.parallel_loop(0, _CH, unroll=2)
        def row_body(r):
            for g in range(_LPR // 8):
                w = [wpe_v[half * _CH + r, pl.ds((g * 8 + j) * 16, 16)]
                     for j in range(8)]
                for j in range(8):
                    plsc.addupdate(
                        tok_v.at[bi, r, pl.ds((g * 8 + j) * 16, 16)], w[j])

    hg = [None] * _NCHUNK
    ho = [None] * _NCHUNK
    hg[0] = gather_start(0, 0)
    hg[1] = gather_start(1, 1)
    wpe_copy.wait()
    for c in range(_NCHUNK):
        bi = c % 3
        if c + 2 < _NCHUNK:
            if c >= 1:
                ho[c - 1].wait()
            hg[c + 2] = gather_start(c + 2, (c + 2) % 3)
        hg[c].wait()
        add_chunk(bi, c % 2)
        ho[c] = store_start(c, bi)
    for c in range(_NCHUNK - 3, _NCHUNK):
        ho[c].wait()


def kernel(x, wte, wpe):
    return _emb_kernel(x.astype(jnp.int32), wte, wpe)


# speedup vs baseline: 1.0484x; 1.0484x over previous
import functools

import jax
import jax.numpy as jnp
from jax import lax
from jax.experimental import pallas as pl
from jax.experimental.pallas import tpu as pltpu
from jax.experimental.pallas import tpu_sc as plsc

_B, _S, _D = 4, 2048, 768
_NC, _NS = 2, 16
_NW = _NC * _NS
_PPW = _S // _NW
_CH = 32
_LPR = _D // 16

_mesh = plsc.VectorSubcoreMesh(core_axis_name="c", subcore_axis_name="s")


@functools.partial(
    pl.kernel,
    mesh=_mesh,
    out_type=jax.ShapeDtypeStruct((_B, _S, _D), jnp.float32),
    scratch_types=[
        pltpu.VMEM((_B, _PPW), jnp.int32),
        pltpu.VMEM((_PPW, _D), jnp.float32),
        pltpu.VMEM((2, _CH, _D), jnp.float32),
        pltpu.SemaphoreType.DMA((2,)),
        pltpu.SemaphoreType.DMA((2,)),
        pltpu.SemaphoreType.DMA((2,)),
    ],
)
def _emb_kernel(x_hbm, wte_hbm, wpe_hbm, out_hbm,
                idx_v, wpe_v, tok_v,
                sem_in, gsems, osems):
    wid = lax.axis_index("s") * _NC + lax.axis_index("c")
    pos0 = wid * _PPW

    idx_copies = [
        pltpu.async_copy(x_hbm.at[b, pl.ds(pos0, _PPW)], idx_v.at[b],
                         sem_in.at[0])
        for b in range(_B)
    ]
    wpe_copy = pltpu.async_copy(
        wpe_hbm.at[pl.ds(pos0, _PPW)], wpe_v, sem_in.at[1])
    for c in idx_copies:
        c.wait()

    def gather_desc(b, half):
        return pltpu.make_async_copy(
            wte_hbm.at[idx_v.at[b, pl.ds(half * _CH, _CH)]],
            tok_v.at[half], gsems.at[half])

    def store_desc(b, half):
        return pltpu.make_async_copy(
            tok_v.at[half],
            out_hbm.at[b, pl.ds(pos0 + half * _CH, _CH)], osems.at[half])

    def add_chunk(half):
        @plsc.parallel_loop(0, _CH, unroll=2)
        def row_body(r):
            for g in range(_LPR // 8):
                w = [wpe_v[half * _CH + r, pl.ds((g * 8 + j) * 16, 16)]
                     for j in range(8)]
                for j in range(8):
                    plsc.addupdate(
                        tok_v.at[half, r, pl.ds((g * 8 + j) * 16, 16)], w[j])

    gather_desc(0, 0).start()
    gather_desc(0, 1).start()
    wpe_copy.wait()

    @pl.loop(0, _B)
    def superstep(b):
        for half in range(2):
            gather_desc(b, half).wait()
            add_chunk(half)
            store_desc(b, half).start()

        @pl.when(b < _B - 1)
        def _prefetch():
            for half in range(2):
                store_desc(b, half).wait()
                gather_desc(b + 1, half).start()

    for half in range(2):
        store_desc(_B - 1, half).wait()


def kernel(x, wte, wpe):
    return _emb_kernel(x.astype(jnp.int32), wte, wpe)


# speedup vs baseline: 1.1484x; 1.0954x over previous
import functools

import jax
import jax.numpy as jnp
from jax import lax
from jax.experimental import pallas as pl
from jax.experimental.pallas import tpu as pltpu
from jax.experimental.pallas import tpu_sc as plsc

_B, _S, _D = 4, 2048, 768
_NC, _NS = 2, 16
_NW = _NC * _NS
_PPW = _S // _NW
_CH = 32
_LPR = _D // 16

_mesh = plsc.VectorSubcoreMesh(core_axis_name="c", subcore_axis_name="s")


@functools.partial(
    pl.kernel,
    mesh=_mesh,
    out_type=jax.ShapeDtypeStruct((_B, _S, _D), jnp.float32),
    scratch_types=[
        pltpu.VMEM((_B, _PPW), jnp.int32),
        pltpu.VMEM((_PPW, _D), jnp.float32),
        pltpu.VMEM((3, _CH, _D), jnp.float32),
        pltpu.SemaphoreType.DMA((2,)),
        pltpu.SemaphoreType.DMA((3,)),
        pltpu.SemaphoreType.DMA((3,)),
    ],
)
def _emb_kernel(x_hbm, wte_hbm, wpe_hbm, out_hbm,
                idx_v, wpe_v, tok_v,
                sem_in, gsems, osems):
    wid = lax.axis_index("s") * _NC + lax.axis_index("c")
    pos0 = wid * _PPW

    idx_copies = [
        pltpu.async_copy(x_hbm.at[b, pl.ds(pos0, _PPW)], idx_v.at[b],
                         sem_in.at[0])
        for b in range(_B)
    ]
    wpe_copy = pltpu.async_copy(
        wpe_hbm.at[pl.ds(pos0, _PPW)], wpe_v, sem_in.at[1])
    for c in idx_copies:
        c.wait()

    _NCHUNK = 2 * _B

    def gather_desc(c):
        b, half, buf = c // 2, c % 2, c % 3
        return pltpu.make_async_copy(
            wte_hbm.at[idx_v.at[b, pl.ds(half * _CH, _CH)]],
            tok_v.at[buf], gsems.at[buf])

    def store_desc(c):
        b, half, buf = c // 2, c % 2, c % 3
        return pltpu.make_async_copy(
            tok_v.at[buf],
            out_hbm.at[b, pl.ds(pos0 + half * _CH, _CH)], osems.at[buf])

    def add_chunk(buf, half):
        @plsc.parallel_loop(0, _CH, unroll=2)
        def row_body(r):
            for g in range(_LPR // 8):
                w = [wpe_v[half * _CH + r, pl.ds((g * 8 + j) * 16, 16)]
                     for j in range(8)]
                for j in range(8):
                    plsc.addupdate(
                        tok_v.at[buf, r, pl.ds((g * 8 + j) * 16, 16)], w[j])

    gather_desc(0).start()
    gather_desc(1).start()
    wpe_copy.wait()

    @pl.loop(0, _NCHUNK)
    def chunk_step(c):
        @pl.when(jnp.logical_and(c >= 1, c < _NCHUNK - 2))
        def _drain():
            store_desc(c - 1).wait()

        @pl.when(c < _NCHUNK - 2)
        def _prefetch():
            gather_desc(c + 2).start()

        gather_desc(c).wait()
        add_chunk(c % 3, c % 2)
        store_desc(c).start()

    for c in range(_NCHUNK - 3, _NCHUNK):
        store_desc(c).wait()


def kernel(x, wte, wpe):
    return _emb_kernel(x.astype(jnp.int32), wte, wpe)


# speedup vs baseline: 1.2817x; 1.1161x over previous
import functools

import jax
import jax.numpy as jnp
from jax import lax
from jax.experimental import pallas as pl
from jax.experimental.pallas import tpu as pltpu
from jax.experimental.pallas import tpu_sc as plsc

_B, _S, _D = 4, 2048, 768
_NC, _NS = 2, 16
_NW = _NC * _NS
_PPW = _S // _NW
_CH = 16
_CPB = _PPW // _CH
_NBUF = 6
_LPR = _D // 16

_mesh = plsc.VectorSubcoreMesh(core_axis_name="c", subcore_axis_name="s")


@functools.partial(
    pl.kernel,
    mesh=_mesh,
    out_type=jax.ShapeDtypeStruct((_B, _S, _D), jnp.float32),
    scratch_types=[
        pltpu.VMEM((_B, _PPW), jnp.int32),
        pltpu.VMEM((_PPW, _D), jnp.float32),
        pltpu.VMEM((_NBUF, _CH, _D), jnp.float32),
        pltpu.SemaphoreType.DMA((3,)),
        pltpu.SemaphoreType.DMA((_NBUF,)),
        pltpu.SemaphoreType.DMA((_NBUF,)),
    ],
)
def _emb_kernel(x_hbm, wte_hbm, wpe_hbm, out_hbm,
                idx_v, wpe_v, tok_v,
                sem_in, gsems, osems):
    wid = lax.axis_index("s") * _NC + lax.axis_index("c")
    pos0 = wid * _PPW

    idx0_copy = pltpu.async_copy(
        x_hbm.at[0, pl.ds(pos0, _PPW)], idx_v.at[0], sem_in.at[0])
    idx_rest = [
        pltpu.async_copy(x_hbm.at[b, pl.ds(pos0, _PPW)], idx_v.at[b],
                         sem_in.at[1])
        for b in range(1, _B)
    ]
    wpe_copy = pltpu.async_copy(
        wpe_hbm.at[pl.ds(pos0, _PPW)], wpe_v, sem_in.at[2])

    _NCHUNK = _B * _CPB
    _PF = 4
    _DR = _PF - 2

    def gather_desc(c):
        b, part, buf = c // _CPB, c % _CPB, c % _NBUF
        return pltpu.make_async_copy(
            wte_hbm.at[idx_v.at[b, pl.ds(part * _CH, _CH)]],
            tok_v.at[buf], gsems.at[buf])

    def store_desc(c):
        b, part, buf = c // _CPB, c % _CPB, c % _NBUF
        return pltpu.make_async_copy(
            tok_v.at[buf],
            out_hbm.at[b, pl.ds(pos0 + part * _CH, _CH)], osems.at[buf])

    def add_chunk(buf, part):
        @plsc.parallel_loop(0, _CH, unroll=2)
        def row_body(r):
            for g in range(_LPR // 8):
                w = [wpe_v[part * _CH + r, pl.ds((g * 8 + j) * 16, 16)]
                     for j in range(8)]
                for j in range(8):
                    plsc.addupdate(
                        tok_v.at[buf, r, pl.ds((g * 8 + j) * 16, 16)], w[j])

    idx0_copy.wait()
    for c in range(_PF):
        gather_desc(c).start()
    for c in idx_rest:
        c.wait()
    wpe_copy.wait()

    @pl.loop(0, _NCHUNK)
    def chunk_step(c):
        @pl.when(jnp.logical_and(c >= _DR, c < _NCHUNK - _PF))
        def _drain():
            store_desc(c - _DR).wait()

        @pl.when(c < _NCHUNK - _PF)
        def _prefetch():
            gather_desc(c + _PF).start()

        gather_desc(c).wait()
        add_chunk(c % _NBUF, c % _CPB)
        store_desc(c).start()

    for c in range(_NCHUNK - _NBUF, _NCHUNK):
        store_desc(c).wait()


def kernel(x, wte, wpe):
    return _emb_kernel(x.astype(jnp.int32), wte, wpe)
